# baseline (speedup 1.0000x reference)
import jax
import jax.numpy as jnp
from jax import lax
from jax.experimental import pallas as pl
from jax.experimental.pallas import tpu as pltpu
from jax.experimental.pallas import tpu_sc as plsc

N = 10000
E = 160000
H = 4
F = 256
ND = 4000
OUT = 256
B = 8192
NEG = 0.2

NC = 2
NS = 16
L = 16
NW = NC * NS

NPAD = 10240
FH = F // NC

CHA = 128
NCH_A = E // CHA
BCB = 32
NBL_B = E // BCB
PPW = B // NW
CHC = 64

_mesh = plsc.VectorSubcoreMesh(core_axis_name="c", subcore_axis_name="s")


def _tc1_body(x_ref, wg_ref, asrc_ref, adst_ref, z_ref, esed_ref):
    x = x_ref[...]
    wg = wg_ref[...]
    z = jnp.dot(x, wg, preferred_element_type=jnp.float32)
    z_ref[...] = z
    cols = []
    for aref in (asrc_ref, adst_ref):
        for h in range(H):
            a = aref[pl.ds(h, 1), :]
            cols.append(jnp.sum(z[:, h * F:(h + 1) * F] * a, axis=1,
                                keepdims=True))
    esed_ref[...] = jnp.concatenate(cols, axis=1)


def _tc1(node_feat, Wg, a_src, a_dst):
    return pl.pallas_call(
        _tc1_body,
        grid=(10,),
        in_specs=[
            pl.BlockSpec((1000, F), lambda i: (i, 0)),
            pl.BlockSpec((F, H * F), lambda i: (0, 0)),
            pl.BlockSpec((H, F), lambda i: (0, 0)),
            pl.BlockSpec((H, F), lambda i: (0, 0)),
        ],
        out_specs=[
            pl.BlockSpec((1000, H * F), lambda i: (i, 0)),
            pl.BlockSpec((1000, 8), lambda i: (i, 0)),
        ],
        out_shape=[
            jax.ShapeDtypeStruct((N, H * F), jnp.float32),
            jax.ShapeDtypeStruct((N, 8), jnp.float32),
        ],
    )(node_feat, Wg, a_src, a_dst)


def _sca_body(esed_hbm, src_hbm, dst_hbm, expf_hbm, denp_hbm,
              srcb, dstb, isrc, idst, idxd, esv, edv, pb, zba, gsem, den_sh):
    cid = lax.axis_index("c")
    sid = lax.axis_index("s")
    wid = sid * NC + cid

    def _zero(i, _):
        zba[pl.ds(i * L, L)] = jnp.zeros((L,), jnp.float32)
        return 0
    lax.fori_loop(0, (4 * NPAD // NS) // L, _zero, 0)
    pltpu.sync_copy(zba, den_sh.at[pl.ds(sid * (4 * NPAD // NS),
                                         4 * NPAD // NS)])
    plsc.subcore_barrier()

    def _do_chunk(cno):
        eoff = cno * CHA
        pltpu.sync_copy(src_hbm.at[pl.ds(eoff, CHA)], srcb)
        pltpu.sync_copy(dst_hbm.at[pl.ds(eoff, CHA)], dstb)
        def _bidx(g, _):
            sv = srcb[pl.ds(g * L, L)]
            dv = dstb[pl.ds(g * L, L)]
            for h in range(H):
                isrc[h, pl.ds(g * L, L)] = sv * 8 + h
                idst[h, pl.ds(g * L, L)] = dv * 8 + (4 + h)
                idxd[h, pl.ds(g * L, L)] = dv * 4 + h
            return 0
        lax.fori_loop(0, CHA // L, _bidx, 0)
        cps = []
        for h in range(H):
            cps.append(pltpu.async_copy(esed_hbm.at[isrc.at[h]], esv.at[h], gsem))
            cps.append(pltpu.async_copy(esed_hbm.at[idst.at[h]], edv.at[h], gsem))
        for cp in cps:
            cp.wait()

        def _grp(g, _):
            for h in range(H):
                e = esv[h, pl.ds(g * L, L)] + edv[h, pl.ds(g * L, L)]
                e = jnp.where(e >= 0, e, NEG * e)
                pb[h, pl.ds(g * L, L)] = jnp.exp(e)
            return 0
        lax.fori_loop(0, CHA // L, _grp, 0)
        for h in range(H):
            pltpu.sync_copy(pb.at[h], expf_hbm.at[pl.ds(h * E + eoff, CHA)])
            pltpu.sync_copy(pb.at[h], den_sh.at[idxd.at[h]], add=True)

    def _chunk(j, _):
        _do_chunk(wid + NW * j)
        return 0
    nfull = NCH_A // NW
    lax.fori_loop(0, nfull, _chunk, 0)
    @pl.when(wid < NCH_A - nfull * NW)
    def _():
        _do_chunk(nfull * NW + wid)

    plsc.subcore_barrier()
    w = 4 * NPAD // NS
    pltpu.sync_copy(den_sh.at[pl.ds(sid * w, w)],
                    denp_hbm.at[pl.ds(cid * 4 * NPAD + sid * w, w)])


def _sca(esed_flat, src, dst):
    w = 4 * NPAD // NS
    f = pl.kernel(
        _sca_body,
        out_type=(
            jax.ShapeDtypeStruct((H * E,), jnp.float32),
            jax.ShapeDtypeStruct((NC * 4 * NPAD,), jnp.float32),
        ),
        mesh=_mesh,
        scratch_types=[
            pltpu.VMEM((CHA,), jnp.int32),
            pltpu.VMEM((CHA,), jnp.int32),
            pltpu.VMEM((H, CHA), jnp.int32),
            pltpu.VMEM((H, CHA), jnp.int32),
            pltpu.VMEM((H, CHA), jnp.int32),
            pltpu.VMEM((H, CHA), jnp.float32),
            pltpu.VMEM((H, CHA), jnp.float32),
            pltpu.VMEM((H, CHA), jnp.float32),
            pltpu.VMEM((w,), jnp.float32),
            pltpu.SemaphoreType.DMA,
            pltpu.VMEM_SHARED((4 * NPAD,), jnp.float32),
        ],
    )
    return f(esed_flat, src, dst)


def _sca2_body(expf_hbm, denp_hbm, dst_hbm, alpha_hbm,
               dstb, idxd, pv, d0, d1, gsem):
    cid = lax.axis_index("c")
    sid = lax.axis_index("s")
    wid = sid * NC + cid

    def _do_chunk(cno):
        eoff = cno * CHA
        pltpu.sync_copy(dst_hbm.at[pl.ds(eoff, CHA)], dstb)
        def _bidx(g, _):
            dv = dstb[pl.ds(g * L, L)]
            for h in range(H):
                idxd[h, pl.ds(g * L, L)] = dv * 4 + h
            return 0
        lax.fori_loop(0, CHA // L, _bidx, 0)
        cps = []
        for h in range(H):
            cps.append(pltpu.async_copy(
                expf_hbm.at[pl.ds(h * E + eoff, CHA)], pv.at[h], gsem))
            cps.append(pltpu.async_copy(denp_hbm.at[idxd.at[h]], d0.at[h], gsem))
        for cp in cps:
            cp.wait()
        def _bidx2(g, _):
            for h in range(H):
                idxd[h, pl.ds(g * L, L)] = idxd[h, pl.ds(g * L, L)] + 4 * NPAD
            return 0
        lax.fori_loop(0, CHA // L, _bidx2, 0)
        cps = [pltpu.async_copy(denp_hbm.at[idxd.at[h]], d1.at[h], gsem)
               for h in range(H)]
        for cp in cps:
            cp.wait()
        def _grp(g, _):
            for h in range(H):
                den = d0[h, pl.ds(g * L, L)] + d1[h, pl.ds(g * L, L)]
                pv[h, pl.ds(g * L, L)] = 0.25 * pv[h, pl.ds(g * L, L)] / den
            return 0
        lax.fori_loop(0, CHA // L, _grp, 0)
        for h in range(H):
            pltpu.sync_copy(pv.at[h], alpha_hbm.at[pl.ds(h * E + eoff, CHA)])

    def _chunk(j, _):
        _do_chunk(wid + NW * j)
        return 0
    nfull = NCH_A // NW
    lax.fori_loop(0, nfull, _chunk, 0)
    @pl.when(wid < NCH_A - nfull * NW)
    def _():
        _do_chunk(nfull * NW + wid)


def _sca2(expf, denp, dst):
    f = pl.kernel(
        _sca2_body,
        out_type=jax.ShapeDtypeStruct((H * E,), jnp.float32),
        mesh=_mesh,
        scratch_types=[
            pltpu.VMEM((CHA,), jnp.int32),
            pltpu.VMEM((H, CHA), jnp.int32),
            pltpu.VMEM((H, CHA), jnp.float32),
            pltpu.VMEM((H, CHA), jnp.float32),
            pltpu.VMEM((H, CHA), jnp.float32),
            pltpu.SemaphoreType.DMA,
        ],
    )
    return f(expf, denp, dst)


def _scb_body(zr_hbm, src_hbm, dst_hbm, arep_hbm, hm_hbm,
              srcb2, dsti2, dscatA, dscatB, idxg2, rows2, arows2, msgA, msgB,
              zb, lsem, gsem0, gsem1, ssem0, ssem1, hacc_sh):
    cid = lax.axis_index("c")
    sid = lax.axis_index("s")
    nb = NBL_B // NS

    def _zb(i, _):
        for j in range(FH // L):
            zb[i, pl.ds(j * L, L)] = jnp.zeros((L,), jnp.float32)
        return 0
    lax.fori_loop(0, 16, _zb, 0)
    for r in range(40):
        pltpu.sync_copy(zb, hacc_sh.at[pl.ds(sid * 640 + r * 16, 16)])
    plsc.subcore_barrier()

    def _lin_cps(jb, p):
        bb = sid + NS * jb
        return (
            pltpu.make_async_copy(src_hbm.at[pl.ds(bb * BCB, BCB)],
                                  srcb2.at[p], lsem),
            pltpu.make_async_copy(dst_hbm.at[pl.ds(bb * BCB, BCB)],
                                  dsti2.at[p], lsem),
            pltpu.make_async_copy(arep_hbm.at[pl.ds(bb * (BCB // 2), BCB // 2)],
                                  arows2.at[p], lsem),
        )

    def fire_lin(jb, p):
        for cp in _lin_cps(jb, p):
            cp.start()

    def wait_lin(jb, p):
        for cp in _lin_cps(jb, p):
            cp.wait()

    def _gat_cp(p):
        sem = gsem0 if p == 0 else gsem1
        return pltpu.make_async_copy(zr_hbm.at[idxg2.at[p]], rows2.at[p], sem)

    def fire_gather(p):
        def _bidx(g, _):
            sv = srcb2[p, pl.ds(g * L, L)]
            for h in range(H):
                idxg2[p, pl.ds(h * BCB + g * L, L)] = sv * 8 + (h * 2 + cid)
            return 0
        lax.fori_loop(0, BCB // L, _bidx, 0)
        _gat_cp(p).start()

    def _sct_cp(p):
        msg = msgA if p == 0 else msgB
        dsc = dscatA if p == 0 else dscatB
        sem = ssem0 if p == 0 else ssem1
        return pltpu.make_async_copy(msg, hacc_sh.at[dsc], sem)

    def compute_scatter(p):
        msg = msgA if p == 0 else msgB
        dsc = dscatA if p == 0 else dscatB
        def _edge(k, _):
            r2 = k // 2
            lo = (k % 2) * 64
            ab0 = arows2[p, r2, pl.ds(lo, L)]
            ab1 = arows2[p, r2, pl.ds(lo + 16, L)]
            ab2 = arows2[p, r2, pl.ds(lo + 32, L)]
            ab3 = arows2[p, r2, pl.ds(lo + 48, L)]
            for j in range(FH // L):
                m = ab0 * rows2[p, k, pl.ds(j * L, L)]
                m = m + ab1 * rows2[p, BCB + k, pl.ds(j * L, L)]
                m = m + ab2 * rows2[p, 2 * BCB + k, pl.ds(j * L, L)]
                m = m + ab3 * rows2[p, 3 * BCB + k, pl.ds(j * L, L)]
                msg[k, pl.ds(j * L, L)] = m
            return 0
        lax.fori_loop(0, BCB, _edge, 0)
        for g in range(BCB // L):
            dsc[pl.ds(g * L, L)] = dsti2[p, pl.ds(g * L, L)]
        sem = ssem0 if p == 0 else ssem1
        pltpu.async_copy(msg, hacc_sh.at[dsc], sem, add=True)

    fire_lin(0, 0)
    wait_lin(0, 0)
    fire_gather(0)
    fire_lin(1, 1)

    def _pair(ji, _):
        jb0 = 2 * ji
        wait_lin(jb0 + 1, 1)
        fire_gather(1)
        _gat_cp(0).wait()
        @pl.when(ji > 0)
        def _():
            _sct_cp(0).wait()
        compute_scatter(0)
        @pl.when(ji < nb // 2 - 1)
        def _():
            fire_lin(jb0 + 2, 0)
        @pl.when(ji < nb // 2 - 1)
        def _():
            wait_lin(jb0 + 2, 0)
            fire_gather(0)
        _gat_cp(1).wait()
        @pl.when(ji > 0)
        def _():
            _sct_cp(1).wait()
        compute_scatter(1)
        @pl.when(ji < nb // 2 - 1)
        def _():
            fire_lin(jb0 + 3, 1)
        return 0
    lax.fori_loop(0, nb // 2, _pair, 0)
    _sct_cp(0).wait()
    _sct_cp(1).wait()

    @pl.when(sid < NBL_B - nb * NS)
    def _():
        fire_lin(nb, 0)
        wait_lin(nb, 0)
        fire_gather(0)
        _gat_cp(0).wait()
        compute_scatter(0)
        _sct_cp(0).wait()

    plsc.subcore_barrier()
    pltpu.sync_copy(hacc_sh.at[pl.ds(sid * 640, 640)],
                    hm_hbm.at[cid, pl.ds(sid * 640, 640)])


def _scb(zr, src, dst, arep):
    f = pl.kernel(
        _scb_body,
        out_type=jax.ShapeDtypeStruct((NC, NPAD, FH), jnp.float32),
        mesh=_mesh,
        scratch_types=[
            pltpu.VMEM((2, BCB), jnp.int32),
            pltpu.VMEM((2, BCB), jnp.int32),
            pltpu.VMEM((BCB,), jnp.int32),
            pltpu.VMEM((BCB,), jnp.int32),
            pltpu.VMEM((2, H * BCB), jnp.int32),
            pltpu.VMEM((2, H * BCB, FH), jnp.float32),
            pltpu.VMEM((2, BCB // 2, FH), jnp.float32),
            pltpu.VMEM((BCB, FH), jnp.float32),
            pltpu.VMEM((BCB, FH), jnp.float32),
            pltpu.VMEM((16, FH), jnp.float32),
            pltpu.SemaphoreType.DMA,
            pltpu.SemaphoreType.DMA,
            pltpu.SemaphoreType.DMA,
            pltpu.SemaphoreType.DMA,
            pltpu.SemaphoreType.DMA,
            pltpu.VMEM_SHARED((NPAD, FH), jnp.float32),
        ],
    )
    return f(zr, src, dst, arep)


def _tc2_body(hm0_ref, hm1_ref, sim_ref, w_ref, b_ref, wdec_ref, h_ref, g_ref):
    w = w_ref[0]
    acc = jnp.dot(hm0_ref[...], w[:FH, :], preferred_element_type=jnp.float32)
    acc += jnp.dot(hm1_ref[...], w[FH:2 * FH, :], preferred_element_type=jnp.float32)
    acc += jnp.dot(sim_ref[...], w[2 * FH:, :], preferred_element_type=jnp.float32)
    acc += b_ref[0][0:1, :]
    h = jnp.where(acc > 0, acc, jnp.exp(jnp.minimum(acc, 0.0)) - 1.0)
    h_ref[...] = h
    g_ref[...] = jnp.dot(h, wdec_ref[...], preferred_element_type=jnp.float32)


def _tc2(hm0, hm1, sim, w_stack, b_stack, W_dec):
    sel3 = lambda i: (lax.min(i // 4, 1), 0, 0)
    return pl.pallas_call(
        _tc2_body,
        grid=(10,),
        in_specs=[
            pl.BlockSpec((1000, FH), lambda i: (i, 0)),
            pl.BlockSpec((1000, FH), lambda i: (i, 0)),
            pl.BlockSpec((1000, FH), lambda i: (i, 0)),
            pl.BlockSpec((1, 3 * FH, OUT), sel3),
            pl.BlockSpec((1, 8, OUT), sel3),
            pl.BlockSpec((OUT, OUT), lambda i: (0, 0)),
        ],
        out_specs=[
            pl.BlockSpec((1000, OUT), lambda i: (i, 0)),
            pl.BlockSpec((1000, OUT), lambda i: (i, 0)),
        ],
        out_shape=[
            jax.ShapeDtypeStruct((N, OUT), jnp.float32),
            jax.ShapeDtypeStruct((N, OUT), jnp.float32),
        ],
    )(hm0, hm1, sim, w_stack, b_stack, W_dec)


def _scc_body(g_hbm, h_hbm, dis_hbm, mir_hbm, gd_hbm, hm_hbm,
              idxd, idxm, gv, hv, gsem):
    cid = lax.axis_index("c")
    sid = lax.axis_index("s")
    wid = sid * NC + cid
    wbase = wid * PPW

    def _chunk(c, _):
        base = wbase + c * CHC
        pltpu.sync_copy(dis_hbm.at[pl.ds(base, CHC)], idxd)
        pltpu.sync_copy(mir_hbm.at[pl.ds(base, CHC)], idxm)
        cg = pltpu.async_copy(g_hbm.at[idxd], gv, gsem)
        ch = pltpu.async_copy(h_hbm.at[idxm], hv, gsem)
        cg.wait()
        ch.wait()
        pltpu.sync_copy(gv, gd_hbm.at[pl.ds(base, CHC)])
        pltpu.sync_copy(hv, hm_hbm.at[pl.ds(base, CHC)])
        return 0
    lax.fori_loop(0, PPW // CHC, _chunk, 0)


def _scc(G, h, diseases, mrnas):
    f = pl.kernel(
        _scc_body,
        out_type=(
            jax.ShapeDtypeStruct((B, OUT), jnp.float32),
            jax.ShapeDtypeStruct((B, OUT), jnp.float32),
        ),
        mesh=_mesh,
        scratch_types=[
            pltpu.VMEM((CHC,), jnp.int32),
            pltpu.VMEM((CHC,), jnp.int32),
            pltpu.VMEM((CHC, OUT), jnp.float32),
            pltpu.VMEM((CHC, OUT), jnp.float32),
            pltpu.SemaphoreType.DMA,
        ],
    )
    return f(G, h, diseases, mrnas)


def _tc3_body(gd_ref, hm_ref, o_ref):
    s = jnp.sum(gd_ref[...] * hm_ref[...], axis=1)
    o_ref[...] = 1.0 / (1.0 + jnp.exp(-s))


def _tc3(Gd, Hm):
    return pl.pallas_call(
        _tc3_body,
        grid=(8,),
        in_specs=[
            pl.BlockSpec((1024, OUT), lambda i: (i, 0)),
            pl.BlockSpec((1024, OUT), lambda i: (i, 0)),
        ],
        out_specs=pl.BlockSpec((1024,), lambda i: (i,)),
        out_shape=jax.ShapeDtypeStruct((B,), jnp.float32),
    )(Gd, Hm)


def kernel(node_feat, d_sim, m_sim, edge_index, diseases, mrnas,
           Wg, a_src, a_dst, m_fc_W, m_fc_b, d_fc_W, d_fc_b, W_dec):
    src = edge_index[0].astype(jnp.int32)
    dst = edge_index[1].astype(jnp.int32)

    z, esed = _tc1(node_feat, Wg, a_src, a_dst)
    zr = z.reshape(N * 8, FH)

    expf, denp = _sca(esed.reshape(-1), src, dst)
    alphaE = _sca2(expf, denp, dst)

    arep = jnp.broadcast_to(
        alphaE.reshape(H, E).T.reshape(E // 2, 8, 1), (E // 2, 8, L)
    ).reshape(E // 2, 8 * L)

    hm = _scb(zr, src, dst, arep)

    sim = jnp.concatenate([d_sim[:ND], m_sim[ND:]], axis=0)
    w_stack = jnp.stack([d_fc_W, m_fc_W])
    b_stack = jnp.broadcast_to(jnp.stack([d_fc_b, m_fc_b])[:, None, :],
                               (2, 8, OUT))

    h, G = _tc2(hm[0, :N], hm[1, :N], sim, w_stack, b_stack, W_dec)
    Gd, Hm = _scc(G, h, diseases.astype(jnp.int32), mrnas.astype(jnp.int32))
    return _tc3(Gd, Hm)


# speedup vs baseline: 1.2065x; 1.2065x over previous
import jax
import jax.numpy as jnp
from jax import lax
from jax.experimental import pallas as pl
from jax.experimental.pallas import tpu as pltpu
from jax.experimental.pallas import tpu_sc as plsc

N = 10000
E = 160000
H = 4
F = 256
ND = 4000
OUT = 256
B = 8192
NEG = 0.2

NC = 2
NS = 16
L = 16
NW = NC * NS

NPAD = 10240
FH = F // NC

CHA = 128
NCH_A = E // CHA
BCB = 32
NBL_B = E // BCB
PPW = B // NW
CHC = 64

_mesh = plsc.VectorSubcoreMesh(core_axis_name="c", subcore_axis_name="s")


def _tc1_body(x_ref, wg_ref, asrc_ref, adst_ref, z_ref, esed_ref):
    x = x_ref[...]
    wg = wg_ref[...]
    z = jnp.dot(x, wg, preferred_element_type=jnp.float32)
    z_ref[...] = z
    cols = []
    for aref in (asrc_ref, adst_ref):
        for h in range(H):
            a = aref[pl.ds(h, 1), :]
            cols.append(jnp.sum(z[:, h * F:(h + 1) * F] * a, axis=1,
                                keepdims=True))
    esed_ref[...] = jnp.concatenate(cols, axis=1)


def _tc1(node_feat, Wg, a_src, a_dst):
    return pl.pallas_call(
        _tc1_body,
        grid=(10,),
        in_specs=[
            pl.BlockSpec((1000, F), lambda i: (i, 0)),
            pl.BlockSpec((F, H * F), lambda i: (0, 0)),
            pl.BlockSpec((H, F), lambda i: (0, 0)),
            pl.BlockSpec((H, F), lambda i: (0, 0)),
        ],
        out_specs=[
            pl.BlockSpec((1000, H * F), lambda i: (i, 0)),
            pl.BlockSpec((1000, 8), lambda i: (i, 0)),
        ],
        out_shape=[
            jax.ShapeDtypeStruct((N, H * F), jnp.float32),
            jax.ShapeDtypeStruct((N, 8), jnp.float32),
        ],
    )(node_feat, Wg, a_src, a_dst)


def _sca_body(esed_hbm, src_hbm, dst_hbm, expf_hbm, denp_hbm,
              srcb, dstb, isrc, idst, idxd, esv, edv, pb, zba, gsem, den_sh):
    cid = lax.axis_index("c")
    sid = lax.axis_index("s")
    wid = sid * NC + cid

    def _zero(i, _):
        zba[pl.ds(i * L, L)] = jnp.zeros((L,), jnp.float32)
        return 0
    lax.fori_loop(0, (4 * NPAD // NS) // L, _zero, 0)
    pltpu.sync_copy(zba, den_sh.at[pl.ds(sid * (4 * NPAD // NS),
                                         4 * NPAD // NS)])
    plsc.subcore_barrier()

    def _do_chunk(cno):
        eoff = cno * CHA
        pltpu.sync_copy(src_hbm.at[pl.ds(eoff, CHA)], srcb)
        pltpu.sync_copy(dst_hbm.at[pl.ds(eoff, CHA)], dstb)
        def _bidx(g, _):
            sv = srcb[pl.ds(g * L, L)]
            dv = dstb[pl.ds(g * L, L)]
            for h in range(H):
                isrc[h, pl.ds(g * L, L)] = sv * 8 + h
                idst[h, pl.ds(g * L, L)] = dv * 8 + (4 + h)
                idxd[h, pl.ds(g * L, L)] = dv * 4 + h
            return 0
        lax.fori_loop(0, CHA // L, _bidx, 0)
        cps = []
        for h in range(H):
            cps.append(pltpu.async_copy(esed_hbm.at[isrc.at[h]], esv.at[h], gsem))
            cps.append(pltpu.async_copy(esed_hbm.at[idst.at[h]], edv.at[h], gsem))
        for cp in cps:
            cp.wait()

        def _grp(g, _):
            for h in range(H):
                e = esv[h, pl.ds(g * L, L)] + edv[h, pl.ds(g * L, L)]
                e = jnp.where(e >= 0, e, NEG * e)
                pb[h, pl.ds(g * L, L)] = jnp.exp(e)
            return 0
        lax.fori_loop(0, CHA // L, _grp, 0)
        for h in range(H):
            pltpu.sync_copy(pb.at[h], expf_hbm.at[pl.ds(h * E + eoff, CHA)])
            pltpu.sync_copy(pb.at[h], den_sh.at[idxd.at[h]], add=True)

    def _chunk(j, _):
        _do_chunk(wid + NW * j)
        return 0
    nfull = NCH_A // NW
    lax.fori_loop(0, nfull, _chunk, 0)
    @pl.when(wid < NCH_A - nfull * NW)
    def _():
        _do_chunk(nfull * NW + wid)

    plsc.subcore_barrier()
    w = 4 * NPAD // NS
    pltpu.sync_copy(den_sh.at[pl.ds(sid * w, w)],
                    denp_hbm.at[pl.ds(cid * 4 * NPAD + sid * w, w)])


def _sca(esed_flat, src, dst):
    w = 4 * NPAD // NS
    f = pl.kernel(
        _sca_body,
        out_type=(
            jax.ShapeDtypeStruct((H * E,), jnp.float32),
            jax.ShapeDtypeStruct((NC * 4 * NPAD,), jnp.float32),
        ),
        mesh=_mesh,
        scratch_types=[
            pltpu.VMEM((CHA,), jnp.int32),
            pltpu.VMEM((CHA,), jnp.int32),
            pltpu.VMEM((H, CHA), jnp.int32),
            pltpu.VMEM((H, CHA), jnp.int32),
            pltpu.VMEM((H, CHA), jnp.int32),
            pltpu.VMEM((H, CHA), jnp.float32),
            pltpu.VMEM((H, CHA), jnp.float32),
            pltpu.VMEM((H, CHA), jnp.float32),
            pltpu.VMEM((w,), jnp.float32),
            pltpu.SemaphoreType.DMA,
            pltpu.VMEM_SHARED((4 * NPAD,), jnp.float32),
        ],
    )
    return f(esed_flat, src, dst)


def _sca2_body(expf_hbm, denp_hbm, dst_hbm, alpha_hbm,
               dstb, idxd, pv, d0, d1, gsem):
    cid = lax.axis_index("c")
    sid = lax.axis_index("s")
    wid = sid * NC + cid

    def _do_chunk(cno):
        eoff = cno * CHA
        pltpu.sync_copy(dst_hbm.at[pl.ds(eoff, CHA)], dstb)
        def _bidx(g, _):
            dv = dstb[pl.ds(g * L, L)]
            for h in range(H):
                idxd[h, pl.ds(g * L, L)] = dv * 4 + h
            return 0
        lax.fori_loop(0, CHA // L, _bidx, 0)
        cps = []
        for h in range(H):
            cps.append(pltpu.async_copy(
                expf_hbm.at[pl.ds(h * E + eoff, CHA)], pv.at[h], gsem))
            cps.append(pltpu.async_copy(denp_hbm.at[idxd.at[h]], d0.at[h], gsem))
        for cp in cps:
            cp.wait()
        def _bidx2(g, _):
            for h in range(H):
                idxd[h, pl.ds(g * L, L)] = idxd[h, pl.ds(g * L, L)] + 4 * NPAD
            return 0
        lax.fori_loop(0, CHA // L, _bidx2, 0)
        cps = [pltpu.async_copy(denp_hbm.at[idxd.at[h]], d1.at[h], gsem)
               for h in range(H)]
        for cp in cps:
            cp.wait()
        def _grp(g, _):
            for h in range(H):
                den = d0[h, pl.ds(g * L, L)] + d1[h, pl.ds(g * L, L)]
                pv[h, pl.ds(g * L, L)] = 0.25 * pv[h, pl.ds(g * L, L)] / den
            return 0
        lax.fori_loop(0, CHA // L, _grp, 0)
        for h in range(H):
            pltpu.sync_copy(pv.at[h], alpha_hbm.at[pl.ds(h * E + eoff, CHA)])

    def _chunk(j, _):
        _do_chunk(wid + NW * j)
        return 0
    nfull = NCH_A // NW
    lax.fori_loop(0, nfull, _chunk, 0)
    @pl.when(wid < NCH_A - nfull * NW)
    def _():
        _do_chunk(nfull * NW + wid)


def _sca2(expf, denp, dst):
    f = pl.kernel(
        _sca2_body,
        out_type=jax.ShapeDtypeStruct((H * E,), jnp.float32),
        mesh=_mesh,
        scratch_types=[
            pltpu.VMEM((CHA,), jnp.int32),
            pltpu.VMEM((H, CHA), jnp.int32),
            pltpu.VMEM((H, CHA), jnp.float32),
            pltpu.VMEM((H, CHA), jnp.float32),
            pltpu.VMEM((H, CHA), jnp.float32),
            pltpu.SemaphoreType.DMA,
        ],
    )
    return f(expf, denp, dst)


def _scb_body(zr_hbm, src_hbm, dst_hbm, arep_hbm, hm_hbm,
              srcb2, dsti2, dscatA, dscatB, idxg2, rows2, arows2, msgA, msgB,
              zb, lsem, gsem0, gsem1, ssem0, ssem1, hacc_sh):
    cid = lax.axis_index("c")
    sid = lax.axis_index("s")
    nb = NBL_B // NS

    def _zb(i, _):
        for j in range(FH // L):
            zb[i, pl.ds(j * L, L)] = jnp.zeros((L,), jnp.float32)
        return 0
    lax.fori_loop(0, 16, _zb, 0)
    for r in range(40):
        pltpu.sync_copy(zb, hacc_sh.at[pl.ds(sid * 640 + r * 16, 16)])
    plsc.subcore_barrier()

    def _lin_cps(jb, p):
        bb = sid + NS * jb
        return (
            pltpu.make_async_copy(src_hbm.at[pl.ds(bb * BCB, BCB)],
                                  srcb2.at[p], lsem),
            pltpu.make_async_copy(dst_hbm.at[pl.ds(bb * BCB, BCB)],
                                  dsti2.at[p], lsem),
            pltpu.make_async_copy(arep_hbm.at[pl.ds(bb * (BCB // 2), BCB // 2)],
                                  arows2.at[p], lsem),
        )

    def fire_lin(jb, p):
        for cp in _lin_cps(jb, p):
            cp.start()

    def wait_lin(jb, p):
        for cp in _lin_cps(jb, p):
            cp.wait()

    def _gat_cp(p):
        sem = gsem0 if p == 0 else gsem1
        return pltpu.make_async_copy(zr_hbm.at[idxg2.at[p]], rows2.at[p], sem)

    def fire_gather(p):
        def _bidx(g, _):
            sv = srcb2[p, pl.ds(g * L, L)]
            for h in range(H):
                idxg2[p, pl.ds(h * BCB + g * L, L)] = sv * 8 + (h * 2 + cid)
            return 0
        lax.fori_loop(0, BCB // L, _bidx, 0)
        _gat_cp(p).start()

    def _sct_cp(p):
        msg = msgA if p == 0 else msgB
        dsc = dscatA if p == 0 else dscatB
        sem = ssem0 if p == 0 else ssem1
        return pltpu.make_async_copy(msg, hacc_sh.at[dsc], sem)

    def compute_scatter(p):
        msg = msgA if p == 0 else msgB
        dsc = dscatA if p == 0 else dscatB
        def _edge4(it, _):
            k0 = it * 4
            r0 = it * 2
            for dk in range(4):
                k = k0 + dk
                r2 = r0 + dk // 2
                lo = (dk % 2) * 64
                ab0 = arows2[p, r2, pl.ds(lo, L)]
                ab1 = arows2[p, r2, pl.ds(lo + 16, L)]
                ab2 = arows2[p, r2, pl.ds(lo + 32, L)]
                ab3 = arows2[p, r2, pl.ds(lo + 48, L)]
                for j in range(FH // L):
                    m = ab0 * rows2[p, k, pl.ds(j * L, L)]
                    m = m + ab1 * rows2[p, BCB + k, pl.ds(j * L, L)]
                    m = m + ab2 * rows2[p, 2 * BCB + k, pl.ds(j * L, L)]
                    m = m + ab3 * rows2[p, 3 * BCB + k, pl.ds(j * L, L)]
                    msg[k, pl.ds(j * L, L)] = m
            return 0
        lax.fori_loop(0, BCB // 4, _edge4, 0)
        for g in range(BCB // L):
            dsc[pl.ds(g * L, L)] = dsti2[p, pl.ds(g * L, L)]
        sem = ssem0 if p == 0 else ssem1
        pltpu.async_copy(msg, hacc_sh.at[dsc], sem, add=True)

    fire_lin(0, 0)
    wait_lin(0, 0)
    fire_gather(0)
    fire_lin(1, 1)

    def _pair(ji, _):
        jb0 = 2 * ji
        wait_lin(jb0 + 1, 1)
        fire_gather(1)
        _gat_cp(0).wait()
        @pl.when(ji > 0)
        def _():
            _sct_cp(0).wait()
        compute_scatter(0)
        @pl.when(ji < nb // 2 - 1)
        def _():
            fire_lin(jb0 + 2, 0)
        @pl.when(ji < nb // 2 - 1)
        def _():
            wait_lin(jb0 + 2, 0)
            fire_gather(0)
        _gat_cp(1).wait()
        @pl.when(ji > 0)
        def _():
            _sct_cp(1).wait()
        compute_scatter(1)
        @pl.when(ji < nb // 2 - 1)
        def _():
            fire_lin(jb0 + 3, 1)
        return 0
    lax.fori_loop(0, nb // 2, _pair, 0)
    _sct_cp(0).wait()
    _sct_cp(1).wait()

    @pl.when(sid < NBL_B - nb * NS)
    def _():
        fire_lin(nb, 0)
        wait_lin(nb, 0)
        fire_gather(0)
        _gat_cp(0).wait()
        compute_scatter(0)
        _sct_cp(0).wait()

    plsc.subcore_barrier()
    pltpu.sync_copy(hacc_sh.at[pl.ds(sid * 640, 640)],
                    hm_hbm.at[cid, pl.ds(sid * 640, 640)])


def _scb(zr, src, dst, arep):
    f = pl.kernel(
        _scb_body,
        out_type=jax.ShapeDtypeStruct((NC, NPAD, FH), jnp.float32),
        mesh=_mesh,
        scratch_types=[
            pltpu.VMEM((2, BCB), jnp.int32),
            pltpu.VMEM((2, BCB), jnp.int32),
            pltpu.VMEM((BCB,), jnp.int32),
            pltpu.VMEM((BCB,), jnp.int32),
            pltpu.VMEM((2, H * BCB), jnp.int32),
            pltpu.VMEM((2, H * BCB, FH), jnp.float32),
            pltpu.VMEM((2, BCB // 2, FH), jnp.float32),
            pltpu.VMEM((BCB, FH), jnp.float32),
            pltpu.VMEM((BCB, FH), jnp.float32),
            pltpu.VMEM((16, FH), jnp.float32),
            pltpu.SemaphoreType.DMA,
            pltpu.SemaphoreType.DMA,
            pltpu.SemaphoreType.DMA,
            pltpu.SemaphoreType.DMA,
            pltpu.SemaphoreType.DMA,
            pltpu.VMEM_SHARED((NPAD, FH), jnp.float32),
        ],
    )
    return f(zr, src, dst, arep)


def _tc2_body(hm0_ref, hm1_ref, sim_ref, w_ref, b_ref, wdec_ref, h_ref, g_ref):
    w = w_ref[0]
    acc = jnp.dot(hm0_ref[...], w[:FH, :], preferred_element_type=jnp.float32)
    acc += jnp.dot(hm1_ref[...], w[FH:2 * FH, :], preferred_element_type=jnp.float32)
    acc += jnp.dot(sim_ref[...], w[2 * FH:, :], preferred_element_type=jnp.float32)
    acc += b_ref[0][0:1, :]
    h = jnp.where(acc > 0, acc, jnp.exp(jnp.minimum(acc, 0.0)) - 1.0)
    h_ref[...] = h
    g_ref[...] = jnp.dot(h, wdec_ref[...], preferred_element_type=jnp.float32)


def _tc2(hm0, hm1, sim, w_stack, b_stack, W_dec):
    sel3 = lambda i: (lax.min(i // 4, 1), 0, 0)
    return pl.pallas_call(
        _tc2_body,
        grid=(10,),
        in_specs=[
            pl.BlockSpec((1000, FH), lambda i: (i, 0)),
            pl.BlockSpec((1000, FH), lambda i: (i, 0)),
            pl.BlockSpec((1000, FH), lambda i: (i, 0)),
            pl.BlockSpec((1, 3 * FH, OUT), sel3),
            pl.BlockSpec((1, 8, OUT), sel3),
            pl.BlockSpec((OUT, OUT), lambda i: (0, 0)),
        ],
        out_specs=[
            pl.BlockSpec((1000, OUT), lambda i: (i, 0)),
            pl.BlockSpec((1000, OUT), lambda i: (i, 0)),
        ],
        out_shape=[
            jax.ShapeDtypeStruct((N, OUT), jnp.float32),
            jax.ShapeDtypeStruct((N, OUT), jnp.float32),
        ],
    )(hm0, hm1, sim, w_stack, b_stack, W_dec)


def _scc_body(g_hbm, h_hbm, dis_hbm, mir_hbm, gd_hbm, hm_hbm,
              idxd, idxm, gv, hv, gsem):
    cid = lax.axis_index("c")
    sid = lax.axis_index("s")
    wid = sid * NC + cid
    wbase = wid * PPW

    def _chunk(c, _):
        base = wbase + c * CHC
        pltpu.sync_copy(dis_hbm.at[pl.ds(base, CHC)], idxd)
        pltpu.sync_copy(mir_hbm.at[pl.ds(base, CHC)], idxm)
        cg = pltpu.async_copy(g_hbm.at[idxd], gv, gsem)
        ch = pltpu.async_copy(h_hbm.at[idxm], hv, gsem)
        cg.wait()
        ch.wait()
        pltpu.sync_copy(gv, gd_hbm.at[pl.ds(base, CHC)])
        pltpu.sync_copy(hv, hm_hbm.at[pl.ds(base, CHC)])
        return 0
    lax.fori_loop(0, PPW // CHC, _chunk, 0)


def _scc(G, h, diseases, mrnas):
    f = pl.kernel(
        _scc_body,
        out_type=(
            jax.ShapeDtypeStruct((B, OUT), jnp.float32),
            jax.ShapeDtypeStruct((B, OUT), jnp.float32),
        ),
        mesh=_mesh,
        scratch_types=[
            pltpu.VMEM((CHC,), jnp.int32),
            pltpu.VMEM((CHC,), jnp.int32),
            pltpu.VMEM((CHC, OUT), jnp.float32),
            pltpu.VMEM((CHC, OUT), jnp.float32),
            pltpu.SemaphoreType.DMA,
        ],
    )
    return f(G, h, diseases, mrnas)


def _tc3_body(gd_ref, hm_ref, o_ref):
    s = jnp.sum(gd_ref[...] * hm_ref[...], axis=1)
    o_ref[...] = 1.0 / (1.0 + jnp.exp(-s))


def _tc3(Gd, Hm):
    return pl.pallas_call(
        _tc3_body,
        grid=(8,),
        in_specs=[
            pl.BlockSpec((1024, OUT), lambda i: (i, 0)),
            pl.BlockSpec((1024, OUT), lambda i: (i, 0)),
        ],
        out_specs=pl.BlockSpec((1024,), lambda i: (i,)),
        out_shape=jax.ShapeDtypeStruct((B,), jnp.float32),
    )(Gd, Hm)


def kernel(node_feat, d_sim, m_sim, edge_index, diseases, mrnas,
           Wg, a_src, a_dst, m_fc_W, m_fc_b, d_fc_W, d_fc_b, W_dec):
    src = edge_index[0].astype(jnp.int32)
    dst = edge_index[1].astype(jnp.int32)

    z, esed = _tc1(node_feat, Wg, a_src, a_dst)
    zr = z.reshape(N * 8, FH)

    expf, denp = _sca(esed.reshape(-1), src, dst)
    alphaE = _sca2(expf, denp, dst)

    arep = jnp.broadcast_to(
        alphaE.reshape(H, E).T.reshape(E // 2, 8, 1), (E // 2, 8, L)
    ).reshape(E // 2, 8 * L)

    hm = _scb(zr, src, dst, arep)

    sim = jnp.concatenate([d_sim[:ND], m_sim[ND:]], axis=0)
    w_stack = jnp.stack([d_fc_W, m_fc_W])
    b_stack = jnp.broadcast_to(jnp.stack([d_fc_b, m_fc_b])[:, None, :],
                               (2, 8, OUT))

    h, G = _tc2(hm[0, :N], hm[1, :N], sim, w_stack, b_stack, W_dec)
    Gd, Hm = _scc(G, h, diseases.astype(jnp.int32), mrnas.astype(jnp.int32))
    return _tc3(Gd, Hm)


# speedup vs baseline: 1.2195x; 1.0108x over previous
import jax
import jax.numpy as jnp
from jax import lax
from jax.experimental import pallas as pl
from jax.experimental.pallas import tpu as pltpu
from jax.experimental.pallas import tpu_sc as plsc

N = 10000
E = 160000
H = 4
F = 256
ND = 4000
OUT = 256
B = 8192
NEG = 0.2

NC = 2
NS = 16
L = 16
NW = NC * NS

NPAD = 10240
FH = F // NC

CHA = 128
NCH_A = E // CHA
BCB = 32
NBL_B = E // BCB
PPW = B // NW
CHC = 64

_mesh = plsc.VectorSubcoreMesh(core_axis_name="c", subcore_axis_name="s")


def _tc1_body(x_ref, wg_ref, asrc_ref, adst_ref, z_ref, esed_ref):
    x = x_ref[...]
    wg = wg_ref[...]
    z = jnp.dot(x, wg, preferred_element_type=jnp.float32)
    z_ref[...] = z
    cols = []
    for aref in (asrc_ref, adst_ref):
        for h in range(H):
            a = aref[pl.ds(h, 1), :]
            cols.append(jnp.sum(z[:, h * F:(h + 1) * F] * a, axis=1,
                                keepdims=True))
    esed_ref[...] = jnp.concatenate(cols, axis=1)


def _tc1(node_feat, Wg, a_src, a_dst):
    return pl.pallas_call(
        _tc1_body,
        grid=(10,),
        in_specs=[
            pl.BlockSpec((1000, F), lambda i: (i, 0)),
            pl.BlockSpec((F, H * F), lambda i: (0, 0)),
            pl.BlockSpec((H, F), lambda i: (0, 0)),
            pl.BlockSpec((H, F), lambda i: (0, 0)),
        ],
        out_specs=[
            pl.BlockSpec((1000, H * F), lambda i: (i, 0)),
            pl.BlockSpec((1000, 8), lambda i: (i, 0)),
        ],
        out_shape=[
            jax.ShapeDtypeStruct((N, H * F), jnp.float32),
            jax.ShapeDtypeStruct((N, 8), jnp.float32),
        ],
    )(node_feat, Wg, a_src, a_dst)


NCHT_A = NCH_A // NS


def _scaf_body(esed_hbm, src_hbm, dst_hbm, alpha_hbm,
               srcb, dstb, isrc, idst, idxd, esv, edv, pb, av, pbig, zba,
               gsem, den_sh):
    cid = lax.axis_index("c")
    sid = lax.axis_index("s")

    def _zero(i, _):
        zba[pl.ds(i * L, L)] = jnp.zeros((L,), jnp.float32)
        return 0
    lax.fori_loop(0, (2 * NPAD // NS) // L, _zero, 0)
    pltpu.sync_copy(zba, den_sh.at[pl.ds(sid * (2 * NPAD // NS),
                                         2 * NPAD // NS)])
    plsc.subcore_barrier()

    def _p1_chunk(j, cno):
        eoff = cno * CHA
        pltpu.sync_copy(src_hbm.at[pl.ds(eoff, CHA)], srcb)
        pltpu.sync_copy(dst_hbm.at[pl.ds(eoff, CHA)], dstb)
        def _bidx(g, _):
            sv = srcb[pl.ds(g * L, L)]
            dv = dstb[pl.ds(g * L, L)]
            for hh in range(2):
                isrc[hh, pl.ds(g * L, L)] = sv * 8 + (2 * cid + hh)
                idst[hh, pl.ds(g * L, L)] = dv * 8 + (4 + 2 * cid + hh)
                idxd[hh, pl.ds(g * L, L)] = dv * 2 + hh
            return 0
        lax.fori_loop(0, CHA // L, _bidx, 0)
        cps = []
        for hh in range(2):
            cps.append(pltpu.async_copy(esed_hbm.at[isrc.at[hh]], esv.at[hh], gsem))
            cps.append(pltpu.async_copy(esed_hbm.at[idst.at[hh]], edv.at[hh], gsem))
        for cp in cps:
            cp.wait()
        def _grp(g, _):
            for hh in range(2):
                e = esv[hh, pl.ds(g * L, L)] + edv[hh, pl.ds(g * L, L)]
                e = jnp.where(e >= 0, e, NEG * e)
                p = jnp.exp(e)
                pb[hh, pl.ds(g * L, L)] = p
                pbig[hh, pl.ds(j * CHA + g * L, L)] = p
            return 0
        lax.fori_loop(0, CHA // L, _grp, 0)
        for hh in range(2):
            pltpu.sync_copy(pb.at[hh], den_sh.at[idxd.at[hh]], add=True)

    def _p1(j, _):
        _p1_chunk(j, sid + NS * j)
        return 0
    lax.fori_loop(0, NCHT_A, _p1, 0)
    @pl.when(sid < NCH_A - NCHT_A * NS)
    def _():
        _p1_chunk(NCHT_A, NCHT_A * NS + sid)

    plsc.subcore_barrier()

    def _p2_chunk(j, cno):
        eoff = cno * CHA
        pltpu.sync_copy(dst_hbm.at[pl.ds(eoff, CHA)], dstb)
        def _bidx(g, _):
            dv = dstb[pl.ds(g * L, L)]
            for hh in range(2):
                idxd[hh, pl.ds(g * L, L)] = dv * 2 + hh
            return 0
        lax.fori_loop(0, CHA // L, _bidx, 0)
        cps = [pltpu.async_copy(den_sh.at[idxd.at[hh]], esv.at[hh], gsem)
               for hh in range(2)]
        for cp in cps:
            cp.wait()
        def _grp(g, _):
            for hh in range(2):
                av[hh, pl.ds(g * L, L)] = (
                    0.25 * pbig[hh, pl.ds(j * CHA + g * L, L)]
                    / esv[hh, pl.ds(g * L, L)])
            return 0
        lax.fori_loop(0, CHA // L, _grp, 0)
        for hh in range(2):
            pltpu.sync_copy(av.at[hh],
                            alpha_hbm.at[pl.ds((2 * cid + hh) * E + eoff, CHA)])

    def _p2(j, _):
        _p2_chunk(j, sid + NS * j)
        return 0
    lax.fori_loop(0, NCHT_A, _p2, 0)
    @pl.when(sid < NCH_A - NCHT_A * NS)
    def _():
        _p2_chunk(NCHT_A, NCHT_A * NS + sid)


def _scaf(esed_flat, src, dst):
    f = pl.kernel(
        _scaf_body,
        out_type=jax.ShapeDtypeStruct((H * E,), jnp.float32),
        mesh=_mesh,
        scratch_types=[
            pltpu.VMEM((CHA,), jnp.int32),
            pltpu.VMEM((CHA,), jnp.int32),
            pltpu.VMEM((2, CHA), jnp.int32),
            pltpu.VMEM((2, CHA), jnp.int32),
            pltpu.VMEM((2, CHA), jnp.int32),
            pltpu.VMEM((2, CHA), jnp.float32),
            pltpu.VMEM((2, CHA), jnp.float32),
            pltpu.VMEM((2, CHA), jnp.float32),
            pltpu.VMEM((2, CHA), jnp.float32),
            pltpu.VMEM((2, (NCHT_A + 1) * CHA), jnp.float32),
            pltpu.VMEM((2 * NPAD // NS,), jnp.float32),
            pltpu.SemaphoreType.DMA,
            pltpu.VMEM_SHARED((2 * NPAD,), jnp.float32),
        ],
    )
    return f(esed_flat, src, dst)


def _scb_body(zr_hbm, src_hbm, dst_hbm, arep_hbm, hm_hbm,
              srcb2, dsti2, dscatA, dscatB, idxg2, rows2, arows2, msgA, msgB,
              zb, lsem, gsem0, gsem1, ssem0, ssem1, hacc_sh):
    cid = lax.axis_index("c")
    sid = lax.axis_index("s")
    nb = NBL_B // NS

    def _zb(i, _):
        for j in range(FH // L):
            zb[i, pl.ds(j * L, L)] = jnp.zeros((L,), jnp.float32)
        return 0
    lax.fori_loop(0, 16, _zb, 0)
    for r in range(40):
        pltpu.sync_copy(zb, hacc_sh.at[pl.ds(sid * 640 + r * 16, 16)])
    plsc.subcore_barrier()

    def _lin_cps(jb, p):
        bb = sid + NS * jb
        return (
            pltpu.make_async_copy(src_hbm.at[pl.ds(bb * BCB, BCB)],
                                  srcb2.at[p], lsem),
            pltpu.make_async_copy(dst_hbm.at[pl.ds(bb * BCB, BCB)],
                                  dsti2.at[p], lsem),
            pltpu.make_async_copy(arep_hbm.at[pl.ds(bb * (BCB // 2), BCB // 2)],
                                  arows2.at[p], lsem),
        )

    def fire_lin(jb, p):
        for cp in _lin_cps(jb, p):
            cp.start()

    def wait_lin(jb, p):
        for cp in _lin_cps(jb, p):
            cp.wait()

    def _gat_cp(p):
        sem = gsem0 if p == 0 else gsem1
        return pltpu.make_async_copy(zr_hbm.at[idxg2.at[p]], rows2.at[p], sem)

    def fire_gather(p):
        def _bidx(g, _):
            sv = srcb2[p, pl.ds(g * L, L)]
            for h in range(H):
                idxg2[p, pl.ds(h * BCB + g * L, L)] = sv * 8 + (h * 2 + cid)
            return 0
        lax.fori_loop(0, BCB // L, _bidx, 0)
        _gat_cp(p).start()

    def _sct_cp(p):
        msg = msgA if p == 0 else msgB
        dsc = dscatA if p == 0 else dscatB
        sem = ssem0 if p == 0 else ssem1
        return pltpu.make_async_copy(msg, hacc_sh.at[dsc], sem)

    def compute_scatter(p):
        msg = msgA if p == 0 else msgB
        dsc = dscatA if p == 0 else dscatB
        def _edge4(it, _):
            k0 = it * 4
            r0 = it * 2
            for dk in range(4):
                k = k0 + dk
                r2 = r0 + dk // 2
                lo = (dk % 2) * 64
                ab0 = arows2[p, r2, pl.ds(lo, L)]
                ab1 = arows2[p, r2, pl.ds(lo + 16, L)]
                ab2 = arows2[p, r2, pl.ds(lo + 32, L)]
                ab3 = arows2[p, r2, pl.ds(lo + 48, L)]
                for j in range(FH // L):
                    m = ab0 * rows2[p, k, pl.ds(j * L, L)]
                    m = m + ab1 * rows2[p, BCB + k, pl.ds(j * L, L)]
                    m = m + ab2 * rows2[p, 2 * BCB + k, pl.ds(j * L, L)]
                    m = m + ab3 * rows2[p, 3 * BCB + k, pl.ds(j * L, L)]
                    msg[k, pl.ds(j * L, L)] = m
            return 0
        lax.fori_loop(0, BCB // 4, _edge4, 0)
        for g in range(BCB // L):
            dsc[pl.ds(g * L, L)] = dsti2[p, pl.ds(g * L, L)]
        sem = ssem0 if p == 0 else ssem1
        pltpu.async_copy(msg, hacc_sh.at[dsc], sem, add=True)

    fire_lin(0, 0)
    wait_lin(0, 0)
    fire_gather(0)
    fire_lin(1, 1)

    def _pair(ji, _):
        jb0 = 2 * ji
        wait_lin(jb0 + 1, 1)
        fire_gather(1)
        _gat_cp(0).wait()
        @pl.when(ji > 0)
        def _():
            _sct_cp(0).wait()
        compute_scatter(0)
        @pl.when(ji < nb // 2 - 1)
        def _():
            fire_lin(jb0 + 2, 0)
        @pl.when(ji < nb // 2 - 1)
        def _():
            wait_lin(jb0 + 2, 0)
            fire_gather(0)
        _gat_cp(1).wait()
        @pl.when(ji > 0)
        def _():
            _sct_cp(1).wait()
        compute_scatter(1)
        @pl.when(ji < nb // 2 - 1)
        def _():
            fire_lin(jb0 + 3, 1)
        return 0
    lax.fori_loop(0, nb // 2, _pair, 0)
    _sct_cp(0).wait()
    _sct_cp(1).wait()

    @pl.when(sid < NBL_B - nb * NS)
    def _():
        fire_lin(nb, 0)
        wait_lin(nb, 0)
        fire_gather(0)
        _gat_cp(0).wait()
        compute_scatter(0)
        _sct_cp(0).wait()

    plsc.subcore_barrier()
    pltpu.sync_copy(hacc_sh.at[pl.ds(sid * 640, 640)],
                    hm_hbm.at[cid, pl.ds(sid * 640, 640)])


def _scb(zr, src, dst, arep):
    f = pl.kernel(
        _scb_body,
        out_type=jax.ShapeDtypeStruct((NC, NPAD, FH), jnp.float32),
        mesh=_mesh,
        scratch_types=[
            pltpu.VMEM((2, BCB), jnp.int32),
            pltpu.VMEM((2, BCB), jnp.int32),
            pltpu.VMEM((BCB,), jnp.int32),
            pltpu.VMEM((BCB,), jnp.int32),
            pltpu.VMEM((2, H * BCB), jnp.int32),
            pltpu.VMEM((2, H * BCB, FH), jnp.float32),
            pltpu.VMEM((2, BCB // 2, FH), jnp.float32),
            pltpu.VMEM((BCB, FH), jnp.float32),
            pltpu.VMEM((BCB, FH), jnp.float32),
            pltpu.VMEM((16, FH), jnp.float32),
            pltpu.SemaphoreType.DMA,
            pltpu.SemaphoreType.DMA,
            pltpu.SemaphoreType.DMA,
            pltpu.SemaphoreType.DMA,
            pltpu.SemaphoreType.DMA,
            pltpu.VMEM_SHARED((NPAD, FH), jnp.float32),
        ],
    )
    return f(zr, src, dst, arep)


def _tc2_body(hm0_ref, hm1_ref, sim_ref, w_ref, b_ref, wdec_ref, h_ref, g_ref):
    w = w_ref[0]
    acc = jnp.dot(hm0_ref[...], w[:FH, :], preferred_element_type=jnp.float32)
    acc += jnp.dot(hm1_ref[...], w[FH:2 * FH, :], preferred_element_type=jnp.float32)
    acc += jnp.dot(sim_ref[...], w[2 * FH:, :], preferred_element_type=jnp.float32)
    acc += b_ref[0][0:1, :]
    h = jnp.where(acc > 0, acc, jnp.exp(jnp.minimum(acc, 0.0)) - 1.0)
    h_ref[...] = h
    g_ref[...] = jnp.dot(h, wdec_ref[...], preferred_element_type=jnp.float32)


def _tc2(hm0, hm1, sim, w_stack, b_stack, W_dec):
    sel3 = lambda i: (lax.min(i // 4, 1), 0, 0)
    return pl.pallas_call(
        _tc2_body,
        grid=(10,),
        in_specs=[
            pl.BlockSpec((1000, FH), lambda i: (i, 0)),
            pl.BlockSpec((1000, FH), lambda i: (i, 0)),
            pl.BlockSpec((1000, FH), lambda i: (i, 0)),
            pl.BlockSpec((1, 3 * FH, OUT), sel3),
            pl.BlockSpec((1, 8, OUT), sel3),
            pl.BlockSpec((OUT, OUT), lambda i: (0, 0)),
        ],
        out_specs=[
            pl.BlockSpec((1000, OUT), lambda i: (i, 0)),
            pl.BlockSpec((1000, OUT), lambda i: (i, 0)),
        ],
        out_shape=[
            jax.ShapeDtypeStruct((N, OUT), jnp.float32),
            jax.ShapeDtypeStruct((N, OUT), jnp.float32),
        ],
    )(hm0, hm1, sim, w_stack, b_stack, W_dec)


def _scc_body(g_hbm, h_hbm, dis_hbm, mir_hbm, gd_hbm, hm_hbm,
              idxd, idxm, gv, hv, gsem):
    cid = lax.axis_index("c")
    sid = lax.axis_index("s")
    wid = sid * NC + cid
    wbase = wid * PPW

    def _chunk(c, _):
        base = wbase + c * CHC
        pltpu.sync_copy(dis_hbm.at[pl.ds(base, CHC)], idxd)
        pltpu.sync_copy(mir_hbm.at[pl.ds(base, CHC)], idxm)
        cg = pltpu.async_copy(g_hbm.at[idxd], gv, gsem)
        ch = pltpu.async_copy(h_hbm.at[idxm], hv, gsem)
        cg.wait()
        ch.wait()
        pltpu.sync_copy(gv, gd_hbm.at[pl.ds(base, CHC)])
        pltpu.sync_copy(hv, hm_hbm.at[pl.ds(base, CHC)])
        return 0
    lax.fori_loop(0, PPW // CHC, _chunk, 0)


def _scc(G, h, diseases, mrnas):
    f = pl.kernel(
        _scc_body,
        out_type=(
            jax.ShapeDtypeStruct((B, OUT), jnp.float32),
            jax.ShapeDtypeStruct((B, OUT), jnp.float32),
        ),
        mesh=_mesh,
        scratch_types=[
            pltpu.VMEM((CHC,), jnp.int32),
            pltpu.VMEM((CHC,), jnp.int32),
            pltpu.VMEM((CHC, OUT), jnp.float32),
            pltpu.VMEM((CHC, OUT), jnp.float32),
            pltpu.SemaphoreType.DMA,
        ],
    )
    return f(G, h, diseases, mrnas)


def _tc3_body(gd_ref, hm_ref, o_ref):
    s = jnp.sum(gd_ref[...] * hm_ref[...], axis=1)
    o_ref[...] = 1.0 / (1.0 + jnp.exp(-s))


def _tc3(Gd, Hm):
    return pl.pallas_call(
        _tc3_body,
        grid=(8,),
        in_specs=[
            pl.BlockSpec((1024, OUT), lambda i: (i, 0)),
            pl.BlockSpec((1024, OUT), lambda i: (i, 0)),
        ],
        out_specs=pl.BlockSpec((1024,), lambda i: (i,)),
        out_shape=jax.ShapeDtypeStruct((B,), jnp.float32),
    )(Gd, Hm)


def kernel(node_feat, d_sim, m_sim, edge_index, diseases, mrnas,
           Wg, a_src, a_dst, m_fc_W, m_fc_b, d_fc_W, d_fc_b, W_dec):
    src = edge_index[0].astype(jnp.int32)
    dst = edge_index[1].astype(jnp.int32)

    z, esed = _tc1(node_feat, Wg, a_src, a_dst)
    zr = z.reshape(N * 8, FH)

    alphaE = _scaf(esed.reshape(-1), src, dst)

    arep = jnp.broadcast_to(
        alphaE.reshape(H, E).T.reshape(E // 2, 8, 1), (E // 2, 8, L)
    ).reshape(E // 2, 8 * L)

    hm = _scb(zr, src, dst, arep)

    sim = jnp.concatenate([d_sim[:ND], m_sim[ND:]], axis=0)
    w_stack = jnp.stack([d_fc_W, m_fc_W])
    b_stack = jnp.broadcast_to(jnp.stack([d_fc_b, m_fc_b])[:, None, :],
                               (2, 8, OUT))

    h, G = _tc2(hm[0, :N], hm[1, :N], sim, w_stack, b_stack, W_dec)
    Gd, Hm = _scc(G, h, diseases.astype(jnp.int32), mrnas.astype(jnp.int32))
    return _tc3(Gd, Hm)


# speedup vs baseline: 1.4053x; 1.1524x over previous
import jax
import jax.numpy as jnp
from jax import lax
from jax.experimental import pallas as pl
from jax.experimental.pallas import tpu as pltpu
from jax.experimental.pallas import tpu_sc as plsc

N = 10000
E = 160000
H = 4
F = 256
ND = 4000
OUT = 256
B = 8192
NEG = 0.2

NC = 2
NS = 16
L = 16
NW = NC * NS

NPAD = 10240
FH = F // NC

CHA = 128
NCH_A = E // CHA
BCB = 32
NBL_B = E // BCB
PPW = B // NW
CHC = 64

_mesh = plsc.VectorSubcoreMesh(core_axis_name="c", subcore_axis_name="s")


def _tc1_body(x_ref, wg_ref, asrc_ref, adst_ref, z_ref, esed_ref):
    x = x_ref[...]
    wg = wg_ref[...]
    z = jnp.dot(x, wg, preferred_element_type=jnp.float32)
    z_ref[...] = z
    cols = []
    for aref in (asrc_ref, adst_ref):
        for h in range(H):
            a = aref[pl.ds(h, 1), :]
            cols.append(jnp.sum(z[:, h * F:(h + 1) * F] * a, axis=1,
                                keepdims=True))
    esed_ref[...] = jnp.concatenate(cols, axis=1)


def _tc1(node_feat, Wg, a_src, a_dst):
    return pl.pallas_call(
        _tc1_body,
        grid=(10,),
        in_specs=[
            pl.BlockSpec((1000, F), lambda i: (i, 0)),
            pl.BlockSpec((F, H * F), lambda i: (0, 0)),
            pl.BlockSpec((H, F), lambda i: (0, 0)),
            pl.BlockSpec((H, F), lambda i: (0, 0)),
        ],
        out_specs=[
            pl.BlockSpec((1000, H * F), lambda i: (i, 0)),
            pl.BlockSpec((1000, 8), lambda i: (i, 0)),
        ],
        out_shape=[
            jax.ShapeDtypeStruct((N, H * F), jnp.float32),
            jax.ShapeDtypeStruct((N, 8), jnp.float32),
        ],
    )(node_feat, Wg, a_src, a_dst)


NCHT_A = NCH_A // NS


def _scaf_body(esed_hbm, src_hbm, dst_hbm, alpha_hbm,
               srcb2, dstb2, isrc2, idst2, idxd2, sidx2, esv2, edv2, pb2,
               av2, pbig, zba,
               lsem, gsA, gsB, ssA, ssB, osA, osB, den_sh):
    cid = lax.axis_index("c")
    sid = lax.axis_index("s")
    nfull = NCHT_A
    nleft = NCH_A - NCHT_A * NS

    def _zero(i, _):
        zba[pl.ds(i * L, L)] = jnp.zeros((L,), jnp.float32)
        return 0
    lax.fori_loop(0, (2 * NPAD // NS) // L, _zero, 0)
    pltpu.sync_copy(zba, den_sh.at[pl.ds(sid * (2 * NPAD // NS),
                                         2 * NPAD // NS)])
    plsc.subcore_barrier()

    def _lin1(j, p):
        eoff = (sid + NS * j) * CHA
        return (pltpu.make_async_copy(src_hbm.at[pl.ds(eoff, CHA)],
                                      srcb2.at[p], lsem),
                pltpu.make_async_copy(dst_hbm.at[pl.ds(eoff, CHA)],
                                      dstb2.at[p], lsem))

    def _gat1(p):
        gs = gsA if p == 0 else gsB
        return ([pltpu.make_async_copy(esed_hbm.at[isrc2.at[p, hh]],
                                       esv2.at[p, hh], gs) for hh in range(2)]
                + [pltpu.make_async_copy(esed_hbm.at[idst2.at[p, hh]],
                                         edv2.at[p, hh], gs) for hh in range(2)])

    def _sct1(p):
        ss = ssA if p == 0 else ssB
        return [pltpu.make_async_copy(pb2.at[p, hh],
                                      den_sh.at[sidx2.at[p, hh]], ss)
                for hh in range(2)]

    def ready_gather1(p):
        def _bidx(g, _):
            sv = srcb2[p, pl.ds(g * L, L)]
            dv = dstb2[p, pl.ds(g * L, L)]
            for hh in range(2):
                isrc2[p, hh, pl.ds(g * L, L)] = sv * 8 + (2 * cid + hh)
                idst2[p, hh, pl.ds(g * L, L)] = dv * 8 + (4 + 2 * cid + hh)
                idxd2[p, hh, pl.ds(g * L, L)] = dv * 2 + hh
            return 0
        lax.fori_loop(0, CHA // L, _bidx, 0)
        for cp in _gat1(p):
            cp.start()

    def process1(j, p):
        for cp in _gat1(p):
            cp.wait()
        def _grp(g, _):
            for hh in range(2):
                e = (esv2[p, hh, pl.ds(g * L, L)] +
                     edv2[p, hh, pl.ds(g * L, L)])
                e = jnp.where(e >= 0, e, NEG * e)
                pv = jnp.exp(e)
                pb2[p, hh, pl.ds(g * L, L)] = pv
                pbig[hh, pl.ds(j * CHA + g * L, L)] = pv
                sidx2[p, hh, pl.ds(g * L, L)] = idxd2[p, hh, pl.ds(g * L, L)]
            return 0
        lax.fori_loop(0, CHA // L, _grp, 0)
        ss = ssA if p == 0 else ssB
        for hh in range(2):
            pltpu.async_copy(pb2.at[p, hh], den_sh.at[sidx2.at[p, hh]], ss,
                             add=True)

    def _p1pair(ji, _):
        jb0 = 2 * ji
        for cp in _lin1(jb0 + 1, 1):
            cp.wait()
        ready_gather1(1)
        @pl.when(ji > 0)
        def _():
            for cp in _sct1(0):
                cp.wait()
        process1(jb0, 0)
        @pl.when(ji < nfull // 2 - 1)
        def _():
            for cp in _lin1(jb0 + 2, 0):
                cp.start()
        @pl.when(ji < nfull // 2 - 1)
        def _():
            for cp in _lin1(jb0 + 2, 0):
                cp.wait()
            ready_gather1(0)
        @pl.when(ji > 0)
        def _():
            for cp in _sct1(1):
                cp.wait()
        process1(jb0 + 1, 1)
        @pl.when(ji < nfull // 2 - 1)
        def _():
            for cp in _lin1(jb0 + 3, 1):
                cp.start()
        return 0

    for cp in _lin1(0, 0):
        cp.start()
    for cp in _lin1(0, 0):
        cp.wait()
    ready_gather1(0)
    for cp in _lin1(1, 1):
        cp.start()
    lax.fori_loop(0, nfull // 2, _p1pair, 0)
    for p in range(2):
        for cp in _sct1(p):
            cp.wait()
    @pl.when(sid < nleft)
    def _():
        eoff = (NCHT_A * NS + sid) * CHA
        pltpu.sync_copy(src_hbm.at[pl.ds(eoff, CHA)], srcb2.at[0])
        pltpu.sync_copy(dst_hbm.at[pl.ds(eoff, CHA)], dstb2.at[0])
        ready_gather1(0)
        process1(NCHT_A, 0)
        for cp in _sct1(0):
            cp.wait()

    plsc.subcore_barrier()

    def _lin2(j, p):
        eoff = (sid + NS * j) * CHA
        return (pltpu.make_async_copy(dst_hbm.at[pl.ds(eoff, CHA)],
                                      dstb2.at[p], lsem),)

    def _gat2(p):
        gs = gsA if p == 0 else gsB
        return [pltpu.make_async_copy(den_sh.at[idxd2.at[p, hh]],
                                      esv2.at[p, hh], gs) for hh in range(2)]

    def _out2(j, p):
        os = osA if p == 0 else osB
        eoff = (sid + NS * j) * CHA
        return [pltpu.make_async_copy(
            av2.at[p, hh], alpha_hbm.at[pl.ds((2 * cid + hh) * E + eoff, CHA)],
            os) for hh in range(2)]

    def ready_gather2(p):
        def _bidx(g, _):
            dv = dstb2[p, pl.ds(g * L, L)]
            for hh in range(2):
                idxd2[p, hh, pl.ds(g * L, L)] = dv * 2 + hh
            return 0
        lax.fori_loop(0, CHA // L, _bidx, 0)
        for cp in _gat2(p):
            cp.start()

    def process2(j, p):
        for cp in _gat2(p):
            cp.wait()
        def _grp(g, _):
            for hh in range(2):
                av2[p, hh, pl.ds(g * L, L)] = (
                    0.25 * pbig[hh, pl.ds(j * CHA + g * L, L)]
                    / esv2[p, hh, pl.ds(g * L, L)])
            return 0
        lax.fori_loop(0, CHA // L, _grp, 0)
        for cp in _out2(j, p):
            cp.start()

    def _p2pair(ji, _):
        jb0 = 2 * ji
        for cp in _lin2(jb0 + 1, 1):
            cp.wait()
        ready_gather2(1)
        @pl.when(ji > 0)
        def _():
            for cp in _out2(jb0 - 2, 0):
                cp.wait()
        process2(jb0, 0)
        @pl.when(ji < nfull // 2 - 1)
        def _():
            for cp in _lin2(jb0 + 2, 0):
                cp.start()
        @pl.when(ji < nfull // 2 - 1)
        def _():
            for cp in _lin2(jb0 + 2, 0):
                cp.wait()
            ready_gather2(0)
        @pl.when(ji > 0)
        def _():
            for cp in _out2(jb0 - 1, 1):
                cp.wait()
        process2(jb0 + 1, 1)
        @pl.when(ji < nfull // 2 - 1)
        def _():
            for cp in _lin2(jb0 + 3, 1):
                cp.start()
        return 0

    for cp in _lin2(0, 0):
        cp.start()
    for cp in _lin2(0, 0):
        cp.wait()
    ready_gather2(0)
    for cp in _lin2(1, 1):
        cp.start()
    lax.fori_loop(0, nfull // 2, _p2pair, 0)
    for p in range(2):
        for cp in _out2(nfull - 2 + p, p):
            cp.wait()
    @pl.when(sid < nleft)
    def _():
        eoff = (NCHT_A * NS + sid) * CHA
        pltpu.sync_copy(dst_hbm.at[pl.ds(eoff, CHA)], dstb2.at[0])
        ready_gather2(0)
        process2(NCHT_A, 0)
        for cp in _out2(NCHT_A, 0):
            cp.wait()


def _scaf(esed_flat, src, dst):
    f = pl.kernel(
        _scaf_body,
        out_type=jax.ShapeDtypeStruct((H * E,), jnp.float32),
        mesh=_mesh,
        scratch_types=[
            pltpu.VMEM((2, CHA), jnp.int32),
            pltpu.VMEM((2, CHA), jnp.int32),
            pltpu.VMEM((2, 2, CHA), jnp.int32),
            pltpu.VMEM((2, 2, CHA), jnp.int32),
            pltpu.VMEM((2, 2, CHA), jnp.int32),
            pltpu.VMEM((2, 2, CHA), jnp.int32),
            pltpu.VMEM((2, 2, CHA), jnp.float32),
            pltpu.VMEM((2, 2, CHA), jnp.float32),
            pltpu.VMEM((2, 2, CHA), jnp.float32),
            pltpu.VMEM((2, 2, CHA), jnp.float32),
            pltpu.VMEM((2, (NCHT_A + 1) * CHA), jnp.float32),
            pltpu.VMEM((2 * NPAD // NS,), jnp.float32),
            pltpu.SemaphoreType.DMA,
            pltpu.SemaphoreType.DMA,
            pltpu.SemaphoreType.DMA,
            pltpu.SemaphoreType.DMA,
            pltpu.SemaphoreType.DMA,
            pltpu.SemaphoreType.DMA,
            pltpu.SemaphoreType.DMA,
            pltpu.VMEM_SHARED((2 * NPAD,), jnp.float32),
        ],
    )
    return f(esed_flat, src, dst)


def _scb_body(zr_hbm, src_hbm, dst_hbm, arep_hbm, hm_hbm,
              srcb2, dsti2, dscatA, dscatB, idxg2, rows2, arows2, msgA, msgB,
              zb, lsem, gsem0, gsem1, ssem0, ssem1, hacc_sh):
    cid = lax.axis_index("c")
    sid = lax.axis_index("s")
    nb = NBL_B // NS

    def _zb(i, _):
        for j in range(FH // L):
            zb[i, pl.ds(j * L, L)] = jnp.zeros((L,), jnp.float32)
        return 0
    lax.fori_loop(0, 16, _zb, 0)
    for r in range(40):
        pltpu.sync_copy(zb, hacc_sh.at[pl.ds(sid * 640 + r * 16, 16)])
    plsc.subcore_barrier()

    def _lin_cps(jb, p):
        bb = sid + NS * jb
        return (
            pltpu.make_async_copy(src_hbm.at[pl.ds(bb * BCB, BCB)],
                                  srcb2.at[p], lsem),
            pltpu.make_async_copy(dst_hbm.at[pl.ds(bb * BCB, BCB)],
                                  dsti2.at[p], lsem),
            pltpu.make_async_copy(arep_hbm.at[pl.ds(bb * (BCB // 2), BCB // 2)],
                                  arows2.at[p], lsem),
        )

    def fire_lin(jb, p):
        for cp in _lin_cps(jb, p):
            cp.start()

    def wait_lin(jb, p):
        for cp in _lin_cps(jb, p):
            cp.wait()

    def _gat_cp(p):
        sem = gsem0 if p == 0 else gsem1
        return pltpu.make_async_copy(zr_hbm.at[idxg2.at[p]], rows2.at[p], sem)

    def fire_gather(p):
        def _bidx(g, _):
            sv = srcb2[p, pl.ds(g * L, L)]
            for h in range(H):
                idxg2[p, pl.ds(h * BCB + g * L, L)] = sv * 8 + (h * 2 + cid)
            return 0
        lax.fori_loop(0, BCB // L, _bidx, 0)
        _gat_cp(p).start()

    def _sct_cp(p):
        msg = msgA if p == 0 else msgB
        dsc = dscatA if p == 0 else dscatB
        sem = ssem0 if p == 0 else ssem1
        return pltpu.make_async_copy(msg, hacc_sh.at[dsc], sem)

    def compute_scatter(p):
        msg = msgA if p == 0 else msgB
        dsc = dscatA if p == 0 else dscatB
        def _edge4(it, _):
            k0 = it * 4
            r0 = it * 2
            for dk in range(4):
                k = k0 + dk
                r2 = r0 + dk // 2
                lo = (dk % 2) * 64
                ab0 = arows2[p, r2, pl.ds(lo, L)]
                ab1 = arows2[p, r2, pl.ds(lo + 16, L)]
                ab2 = arows2[p, r2, pl.ds(lo + 32, L)]
                ab3 = arows2[p, r2, pl.ds(lo + 48, L)]
                for j in range(FH // L):
                    m = ab0 * rows2[p, k, pl.ds(j * L, L)]
                    m = m + ab1 * rows2[p, BCB + k, pl.ds(j * L, L)]
                    m = m + ab2 * rows2[p, 2 * BCB + k, pl.ds(j * L, L)]
                    m = m + ab3 * rows2[p, 3 * BCB + k, pl.ds(j * L, L)]
                    msg[k, pl.ds(j * L, L)] = m
            return 0
        lax.fori_loop(0, BCB // 4, _edge4, 0)
        for g in range(BCB // L):
            dsc[pl.ds(g * L, L)] = dsti2[p, pl.ds(g * L, L)]
        sem = ssem0 if p == 0 else ssem1
        pltpu.async_copy(msg, hacc_sh.at[dsc], sem, add=True)

    fire_lin(0, 0)
    wait_lin(0, 0)
    fire_gather(0)
    fire_lin(1, 1)

    def _pair(ji, _):
        jb0 = 2 * ji
        wait_lin(jb0 + 1, 1)
        fire_gather(1)
        _gat_cp(0).wait()
        @pl.when(ji > 0)
        def _():
            _sct_cp(0).wait()
        compute_scatter(0)
        @pl.when(ji < nb // 2 - 1)
        def _():
            fire_lin(jb0 + 2, 0)
        @pl.when(ji < nb // 2 - 1)
        def _():
            wait_lin(jb0 + 2, 0)
            fire_gather(0)
        _gat_cp(1).wait()
        @pl.when(ji > 0)
        def _():
            _sct_cp(1).wait()
        compute_scatter(1)
        @pl.when(ji < nb // 2 - 1)
        def _():
            fire_lin(jb0 + 3, 1)
        return 0
    lax.fori_loop(0, nb // 2, _pair, 0)
    _sct_cp(0).wait()
    _sct_cp(1).wait()

    @pl.when(sid < NBL_B - nb * NS)
    def _():
        fire_lin(nb, 0)
        wait_lin(nb, 0)
        fire_gather(0)
        _gat_cp(0).wait()
        compute_scatter(0)
        _sct_cp(0).wait()

    plsc.subcore_barrier()
    pltpu.sync_copy(hacc_sh.at[pl.ds(sid * 640, 640)],
                    hm_hbm.at[cid, pl.ds(sid * 640, 640)])


def _scb(zr, src, dst, arep):
    f = pl.kernel(
        _scb_body,
        out_type=jax.ShapeDtypeStruct((NC, NPAD, FH), jnp.float32),
        mesh=_mesh,
        scratch_types=[
            pltpu.VMEM((2, BCB), jnp.int32),
            pltpu.VMEM((2, BCB), jnp.int32),
            pltpu.VMEM((BCB,), jnp.int32),
            pltpu.VMEM((BCB,), jnp.int32),
            pltpu.VMEM((2, H * BCB), jnp.int32),
            pltpu.VMEM((2, H * BCB, FH), jnp.float32),
            pltpu.VMEM((2, BCB // 2, FH), jnp.float32),
            pltpu.VMEM((BCB, FH), jnp.float32),
            pltpu.VMEM((BCB, FH), jnp.float32),
            pltpu.VMEM((16, FH), jnp.float32),
            pltpu.SemaphoreType.DMA,
            pltpu.SemaphoreType.DMA,
            pltpu.SemaphoreType.DMA,
            pltpu.SemaphoreType.DMA,
            pltpu.SemaphoreType.DMA,
            pltpu.VMEM_SHARED((NPAD, FH), jnp.float32),
        ],
    )
    return f(zr, src, dst, arep)


def _tc2_body(hm0_ref, hm1_ref, sim_ref, w_ref, b_ref, wdec_ref, h_ref, g_ref):
    w = w_ref[0]
    acc = jnp.dot(hm0_ref[...], w[:FH, :], preferred_element_type=jnp.float32)
    acc += jnp.dot(hm1_ref[...], w[FH:2 * FH, :], preferred_element_type=jnp.float32)
    acc += jnp.dot(sim_ref[...], w[2 * FH:, :], preferred_element_type=jnp.float32)
    acc += b_ref[0][0:1, :]
    h = jnp.where(acc > 0, acc, jnp.exp(jnp.minimum(acc, 0.0)) - 1.0)
    h_ref[...] = h
    g_ref[...] = jnp.dot(h, wdec_ref[...], preferred_element_type=jnp.float32)


def _tc2(hm0, hm1, sim, w_stack, b_stack, W_dec):
    sel3 = lambda i: (lax.min(i // 4, 1), 0, 0)
    return pl.pallas_call(
        _tc2_body,
        grid=(10,),
        in_specs=[
            pl.BlockSpec((1000, FH), lambda i: (i, 0)),
            pl.BlockSpec((1000, FH), lambda i: (i, 0)),
            pl.BlockSpec((1000, FH), lambda i: (i, 0)),
            pl.BlockSpec((1, 3 * FH, OUT), sel3),
            pl.BlockSpec((1, 8, OUT), sel3),
            pl.BlockSpec((OUT, OUT), lambda i: (0, 0)),
        ],
        out_specs=[
            pl.BlockSpec((1000, OUT), lambda i: (i, 0)),
            pl.BlockSpec((1000, OUT), lambda i: (i, 0)),
        ],
        out_shape=[
            jax.ShapeDtypeStruct((N, OUT), jnp.float32),
            jax.ShapeDtypeStruct((N, OUT), jnp.float32),
        ],
    )(hm0, hm1, sim, w_stack, b_stack, W_dec)


def _scc_body(g_hbm, h_hbm, dis_hbm, mir_hbm, gd_hbm, hm_hbm,
              idxd, idxm, gv, hv, gsem):
    cid = lax.axis_index("c")
    sid = lax.axis_index("s")
    wid = sid * NC + cid
    wbase = wid * PPW

    def _chunk(c, _):
        base = wbase + c * CHC
        pltpu.sync_copy(dis_hbm.at[pl.ds(base, CHC)], idxd)
        pltpu.sync_copy(mir_hbm.at[pl.ds(base, CHC)], idxm)
        cg = pltpu.async_copy(g_hbm.at[idxd], gv, gsem)
        ch = pltpu.async_copy(h_hbm.at[idxm], hv, gsem)
        cg.wait()
        ch.wait()
        pltpu.sync_copy(gv, gd_hbm.at[pl.ds(base, CHC)])
        pltpu.sync_copy(hv, hm_hbm.at[pl.ds(base, CHC)])
        return 0
    lax.fori_loop(0, PPW // CHC, _chunk, 0)


def _scc(G, h, diseases, mrnas):
    f = pl.kernel(
        _scc_body,
        out_type=(
            jax.ShapeDtypeStruct((B, OUT), jnp.float32),
            jax.ShapeDtypeStruct((B, OUT), jnp.float32),
        ),
        mesh=_mesh,
        scratch_types=[
            pltpu.VMEM((CHC,), jnp.int32),
            pltpu.VMEM((CHC,), jnp.int32),
            pltpu.VMEM((CHC, OUT), jnp.float32),
            pltpu.VMEM((CHC, OUT), jnp.float32),
            pltpu.SemaphoreType.DMA,
        ],
    )
    return f(G, h, diseases, mrnas)


def _tc3_body(gd_ref, hm_ref, o_ref):
    s = jnp.sum(gd_ref[...] * hm_ref[...], axis=1)
    o_ref[...] = 1.0 / (1.0 + jnp.exp(-s))


def _tc3(Gd, Hm):
    return pl.pallas_call(
        _tc3_body,
        grid=(8,),
        in_specs=[
            pl.BlockSpec((1024, OUT), lambda i: (i, 0)),
            pl.BlockSpec((1024, OUT), lambda i: (i, 0)),
        ],
        out_specs=pl.BlockSpec((1024,), lambda i: (i,)),
        out_shape=jax.ShapeDtypeStruct((B,), jnp.float32),
    )(Gd, Hm)


def kernel(node_feat, d_sim, m_sim, edge_index, diseases, mrnas,
           Wg, a_src, a_dst, m_fc_W, m_fc_b, d_fc_W, d_fc_b, W_dec):
    src = edge_index[0].astype(jnp.int32)
    dst = edge_index[1].astype(jnp.int32)

    z, esed = _tc1(node_feat, Wg, a_src, a_dst)
    zr = z.reshape(N * 8, FH)

    alphaE = _scaf(esed.reshape(-1), src, dst)

    arep = jnp.broadcast_to(
        alphaE.reshape(H, E).T.reshape(E // 2, 8, 1), (E // 2, 8, L)
    ).reshape(E // 2, 8 * L)

    hm = _scb(zr, src, dst, arep)

    sim = jnp.concatenate([d_sim[:ND], m_sim[ND:]], axis=0)
    w_stack = jnp.stack([d_fc_W, m_fc_W])
    b_stack = jnp.broadcast_to(jnp.stack([d_fc_b, m_fc_b])[:, None, :],
                               (2, 8, OUT))

    h, G = _tc2(hm[0, :N], hm[1, :N], sim, w_stack, b_stack, W_dec)
    Gd, Hm = _scc(G, h, diseases.astype(jnp.int32), mrnas.astype(jnp.int32))
    return _tc3(Gd, Hm)
